# Initial kernel scaffold; baseline (speedup 1.0000x reference)
import jax
import jax.numpy as jnp
from jax.experimental import pallas as pl


def kernel(q, k, v, iq, ik, w):
    raise NotImplementedError("write your pallas kernel here")


# speedup vs baseline: 18.7081x; 18.7081x over previous
import functools

import jax
import jax.numpy as jnp
from jax.experimental import pallas as pl
from jax.experimental.pallas import tpu as pltpu

B, H, S, DH = 1, 16, 2048, 128
HI, DI = 4, 64
TOPK = 512

BQ = 256
BK = 512
NEG = -1e30


def _mask_kernel(w_ref, iq_ref, ik_ref, mask_ref):
    i = pl.program_id(0)
    prods = []
    for h in range(HI):
        lg = jax.lax.dot_general(
            iq_ref[h], ik_ref[...],
            (((1,), (1,)), ((), ())),
            preferred_element_type=jnp.float32,
        )
        wb = w_ref[h].astype(jnp.bfloat16).astype(jnp.float32)
        lb = jnp.maximum(lg, 0.0).astype(jnp.bfloat16).astype(jnp.float32)
        prods.append(wb * lb)
    acc = prods[0]
    comp = jnp.zeros((BQ, S), dtype=jnp.float32)
    for h in range(1, HI):
        p = prods[h]
        t = acc + p
        big = jnp.abs(acc) >= jnp.abs(p)
        comp = comp + jnp.where(big, (acc - t) + p, (p - t) + acc)
        acc = t
    acc = acc + comp
    s_ix = jax.lax.broadcasted_iota(jnp.int32, (BQ, S), 1)
    t_ix = i * BQ + jax.lax.broadcasted_iota(jnp.int32, (BQ, S), 0)
    sc = jnp.where(s_ix <= t_ix, acc, -1.0)

    th_u = jnp.zeros((BQ, 1), dtype=jnp.int32)
    for bit in range(30, -1, -1):
        trial = th_u | (1 << bit)
        trial_f = jax.lax.bitcast_convert_type(trial, jnp.float32)
        cnt = jnp.sum((sc >= trial_f).astype(jnp.int32), axis=1,
                      keepdims=True)
        th_u = jnp.where(cnt >= TOPK, trial, th_u)
    th_f = jax.lax.bitcast_convert_type(th_u, jnp.float32)

    gt = sc > th_f
    eq = sc == th_f
    c = jnp.sum(gt.astype(jnp.int32), axis=1, keepdims=True)
    quota = TOPK - c

    eq_i = eq.astype(jnp.int32)
    r = jnp.zeros((BQ, 1), dtype=jnp.int32)
    for bit in range(10, -1, -1):
        trial = r | (1 << bit)
        pref = jnp.sum(jnp.where(s_ix <= trial, eq_i, 0), axis=1,
                       keepdims=True)
        r = jnp.where(pref <= quota, trial, r)
    pref0 = jnp.sum(jnp.where(s_ix <= r, eq_i, 0), axis=1, keepdims=True)
    r = jnp.where(pref0 <= quota, r, -1)

    mask = gt | (eq & (s_ix <= r))
    mask_ref[...] = mask.astype(jnp.int8)


def _attn_kernel(q_ref, k_ref, v_ref, mask_ref, o_ref):
    i = pl.program_id(1)
    q = q_ref[0] * (1.0 / (DH ** 0.5))
    nj = i * BQ // BK + 1

    def body(j, carry):
        m, l, acc = carry
        kk = k_ref[0, pl.ds(j * BK, BK), :]
        vv = v_ref[0, pl.ds(j * BK, BK), :]
        msk = mask_ref[:, pl.ds(j * BK, BK)] != 0
        lg = jax.lax.dot_general(
            q, kk, (((1,), (1,)), ((), ())),
            preferred_element_type=jnp.float32,
        )
        lg = jnp.where(msk, lg, NEG)
        m_new = jnp.maximum(m, jnp.max(lg, axis=1, keepdims=True))
        p = jnp.exp(lg - m_new)
        corr = jnp.exp(m - m_new)
        l_new = l * corr + jnp.sum(p, axis=1, keepdims=True)
        acc_new = acc * corr + jax.lax.dot_general(
            p, vv, (((1,), (0,)), ((), ())),
            preferred_element_type=jnp.float32,
        )
        return m_new, l_new, acc_new

    m0 = jnp.full((BQ, 1), NEG, dtype=jnp.float32)
    l0 = jnp.zeros((BQ, 1), dtype=jnp.float32)
    a0 = jnp.zeros((BQ, DH), dtype=jnp.float32)
    m, l, acc = jax.lax.fori_loop(0, nj, body, (m0, l0, a0))
    o_ref[0] = acc / l


@jax.jit
def kernel(q, k, v, iq, ik, w):
    iq3 = iq[0]
    ik2 = ik[0]
    q3, k3, v3 = q[0], k[0], v[0]

    mask = pl.pallas_call(
        _mask_kernel,
        grid=(S // BQ,),
        in_specs=[
            pl.BlockSpec(memory_space=pltpu.SMEM),
            pl.BlockSpec((HI, BQ, DI), lambda i: (0, i, 0)),
            pl.BlockSpec((S, DI), lambda i: (0, 0)),
        ],
        out_specs=pl.BlockSpec((BQ, S), lambda i: (i, 0)),
        out_shape=jax.ShapeDtypeStruct((S, S), jnp.int8),
    )(w, iq3, ik2)

    out = pl.pallas_call(
        _attn_kernel,
        grid=(H, S // BQ),
        in_specs=[
            pl.BlockSpec((1, BQ, DH), lambda h, i: (h, i, 0)),
            pl.BlockSpec((1, S, DH), lambda h, i: (h, 0, 0)),
            pl.BlockSpec((1, S, DH), lambda h, i: (h, 0, 0)),
            pl.BlockSpec((BQ, S), lambda h, i: (i, 0)),
        ],
        out_specs=pl.BlockSpec((1, BQ, DH), lambda h, i: (h, i, 0)),
        out_shape=jax.ShapeDtypeStruct((H, S, DH), jnp.float32),
    )(q3, k3, v3, mask)

    return out[None]
